# baseline (device time: 46730 ns/iter reference)
import jax
import jax.numpy as jnp
from jax import lax
from jax.experimental import pallas as pl
from jax.experimental.pallas import tpu as pltpu

N_DEV = 4
R, L = 0, 1
S = 2


def kernel(x, W1, W2):
    m, k = x.shape
    _, d = W1.shape
    _, f = W2.shape
    chunk = m // N_DEV
    d2 = d // 2
    w = d2 // S

    def body(x_ref, w1_ref, w2_ref, out_ref,
             h_ref, rs_comm, ag_comm,
             rs_send, rs_recv, ag_send, ag_recv):
        my = lax.axis_index("i")
        left = lax.rem(my + N_DEV - 1, N_DEV)
        right = lax.rem(my + 1, N_DEV)

        def mod(e):
            return lax.rem(e + N_DEV, N_DEV)

        def rows(c):
            return pl.ds(c * chunk, chunk)

        def cols(d_, j):
            return pl.ds((0 if d_ == R else d2) + j * w, w)

        barrier_sem = pltpu.get_barrier_semaphore()
        for nbr in (left, right):
            pl.semaphore_signal(
                barrier_sem, inc=1,
                device_id=(nbr,), device_id_type=pl.DeviceIdType.MESH,
            )
        pl.semaphore_wait(barrier_sem, 2)

        def compute_h(c):
            h_ref[rows(c), :] = jnp.dot(
                x_ref[rows(c), :], w1_ref[...],
                preferred_element_type=jnp.float32,
            )

        def compute_out(c):
            out_ref[rows(c), :] = jnp.dot(
                h_ref[rows(c), :], w2_ref[...],
                preferred_element_type=jnp.float32,
            )

        def make_rs(d_, s, j, src_c):
            return pltpu.make_async_remote_copy(
                src_ref=h_ref.at[rows(src_c), cols(d_, j)],
                dst_ref=rs_comm.at[d_, s, j],
                send_sem=rs_send.at[d_, s, j],
                recv_sem=rs_recv.at[d_, s, j],
                device_id=(right if d_ == R else left,),
                device_id_type=pl.DeviceIdType.MESH,
            )

        def make_ag(d_, t, j, src):
            return pltpu.make_async_remote_copy(
                src_ref=src,
                dst_ref=ag_comm.at[d_, t, j],
                send_sem=ag_send.at[d_, t, j],
                recv_sem=ag_recv.at[d_, t, j],
                device_id=(right if d_ == R else left,),
                device_id_type=pl.DeviceIdType.MESH,
            )

        compute_h(my)
        inflight = {}
        for j in range(S):
            for d_ in (R, L):
                rd = make_rs(d_, 0, j, my)
                rd.start()
                inflight[(d_, 0, j)] = rd
        compute_h(mod(my + 1))
        compute_h(mod(my - 1))

        for s in range(1, N_DEV - 1):
            recv_r = mod(my - s)
            recv_l = mod(my + s)
            for j in range(S):
                for d_, rc in ((R, recv_r), (L, recv_l)):
                    inflight[(d_, s - 1, j)].wait_recv()
                    h_ref[rows(rc), cols(d_, j)] = (
                        h_ref[rows(rc), cols(d_, j)] + rs_comm[d_, s - 1, j]
                    )
                    rd = make_rs(d_, s, j, rc)
                    rd.start()
                    inflight[(d_, s, j)] = rd
            if s == 1:
                compute_h(mod(my + 2))

        red_r = mod(my + 1)
        red_l = mod(my - 1)
        ag_inflight = {}
        for j in range(S):
            for d_, rc in ((R, red_r), (L, red_l)):
                inflight[(d_, N_DEV - 2, j)].wait_recv()
                h_ref[rows(rc), cols(d_, j)] = (
                    h_ref[rows(rc), cols(d_, j)] + rs_comm[d_, N_DEV - 2, j]
                )
                rd = make_ag(d_, 0, j, h_ref.at[rows(rc), cols(d_, j)])
                rd.start()
                ag_inflight[(d_, 0, j)] = rd

        for t in range(1, N_DEV - 1):
            got_r = mod(my - (t - 1))
            got_l = mod(my + (t - 1))
            for j in range(S):
                for d_, gc in ((R, got_r), (L, got_l)):
                    ag_inflight[(d_, t - 1, j)].wait_recv()
                    rd = make_ag(d_, t, j, ag_comm.at[d_, t - 1, j])
                    rd.start()
                    ag_inflight[(d_, t, j)] = rd
                    h_ref[rows(gc), cols(d_, j)] = ag_comm[d_, t - 1, j]
            if t == 1:
                compute_out(my)
            else:
                compute_out(mod(my + 1))
                compute_out(mod(my - 1))

        got_r = mod(my - 2)
        got_l = mod(my + 2)
        for j in range(S):
            for d_, gc in ((R, got_r), (L, got_l)):
                ag_inflight[(d_, N_DEV - 2, j)].wait_recv()
                h_ref[rows(gc), cols(d_, j)] = ag_comm[d_, N_DEV - 2, j]
        compute_out(mod(my + 2))

        for rd in inflight.values():
            rd.wait_send()
        for rd in ag_inflight.values():
            rd.wait_send()

    return pl.pallas_call(
        body,
        out_shape=jax.ShapeDtypeStruct((m, f), jnp.float32),
        in_specs=[
            pl.BlockSpec(memory_space=pltpu.VMEM),
            pl.BlockSpec(memory_space=pltpu.VMEM),
            pl.BlockSpec(memory_space=pltpu.VMEM),
        ],
        out_specs=pl.BlockSpec(memory_space=pltpu.VMEM),
        scratch_shapes=[
            pltpu.VMEM((m, d), jnp.float32),
            pltpu.VMEM((2, N_DEV - 1, S, chunk, d // 2 // S), jnp.float32),
            pltpu.VMEM((2, N_DEV - 1, S, chunk, d // 2 // S), jnp.float32),
            pltpu.SemaphoreType.DMA((2, N_DEV - 1, S)),
            pltpu.SemaphoreType.DMA((2, N_DEV - 1, S)),
            pltpu.SemaphoreType.DMA((2, N_DEV - 1, S)),
            pltpu.SemaphoreType.DMA((2, N_DEV - 1, S)),
        ],
        compiler_params=pltpu.CompilerParams(collective_id=0),
    )(x, W1, W2)


# device time: 45107 ns/iter; 1.0360x vs baseline; 1.0360x over previous
import jax
import jax.numpy as jnp
from jax import lax
from jax.experimental import pallas as pl
from jax.experimental.pallas import tpu as pltpu

N_DEV = 4
R, L = 0, 1
S = 2


def kernel(x, W1, W2):
    m, k = x.shape
    _, d = W1.shape
    _, f = W2.shape
    chunk = m // N_DEV
    half = chunk // 2
    rh = half // S

    def body(x_ref, w1_ref, w2_ref, out_ref,
             h_ref, rs_comm, ag_comm,
             rs_send, rs_recv, ag_send, ag_recv):
        my = lax.axis_index("i")
        left = lax.rem(my + N_DEV - 1, N_DEV)
        right = lax.rem(my + 1, N_DEV)

        def mod(e):
            return lax.rem(e + N_DEV, N_DEV)

        def srows(c, d_, j):
            return pl.ds(c * chunk + (0 if d_ == R else half) + j * rh, rh)

        def hrows(c, d_):
            return pl.ds(c * chunk + (0 if d_ == R else half), half)

        def compute_h_half(c, d_):
            h_ref[hrows(c, d_), :] = jnp.dot(
                x_ref[hrows(c, d_), :], w1_ref[...],
                preferred_element_type=jnp.float32,
            )

        def compute_out_rows(r):
            out_ref[r, :] = jnp.dot(
                h_ref[r, :], w2_ref[...],
                preferred_element_type=jnp.float32,
            )

        def make_rs(d_, s, j, src_c):
            return pltpu.make_async_remote_copy(
                src_ref=h_ref.at[srows(src_c, d_, j), :],
                dst_ref=rs_comm.at[d_, s, j],
                send_sem=rs_send.at[d_, s, j],
                recv_sem=rs_recv.at[d_, s, j],
                device_id=(right if d_ == R else left,),
                device_id_type=pl.DeviceIdType.MESH,
            )

        def make_ag(d_, t, j, src):
            return pltpu.make_async_remote_copy(
                src_ref=src,
                dst_ref=ag_comm.at[d_, t, j],
                send_sem=ag_send.at[d_, t, j],
                recv_sem=ag_recv.at[d_, t, j],
                device_id=(right if d_ == R else left,),
                device_id_type=pl.DeviceIdType.MESH,
            )

        compute_h_half(my, R)
        compute_h_half(my, L)
        compute_h_half(mod(my - 1), R)
        compute_h_half(mod(my + 1), L)

        barrier_sem = pltpu.get_barrier_semaphore()
        for nbr in (left, right):
            pl.semaphore_signal(
                barrier_sem, inc=1,
                device_id=(nbr,), device_id_type=pl.DeviceIdType.MESH,
            )
        pl.semaphore_wait(barrier_sem, 2)

        inflight = {}
        for d_ in (R, L):
            for j in range(S):
                rd = make_rs(d_, 0, j, my)
                rd.start()
                inflight[(d_, 0, j)] = rd
        compute_h_half(mod(my - 2), R)
        compute_h_half(mod(my + 2), L)

        for s in range(1, N_DEV - 1):
            recv_r = mod(my - s)
            recv_l = mod(my + s)
            for j in range(S):
                for d_, rc in ((R, recv_r), (L, recv_l)):
                    inflight[(d_, s - 1, j)].wait_recv()
                    h_ref[srows(rc, d_, j), :] = (
                        h_ref[srows(rc, d_, j), :] + rs_comm[d_, s - 1, j]
                    )
                    rd = make_rs(d_, s, j, rc)
                    rd.start()
                    inflight[(d_, s, j)] = rd
            if s == 1:
                compute_h_half(mod(my + 1), R)
                compute_h_half(mod(my - 1), L)

        red_r = mod(my + 1)
        red_l = mod(my - 1)
        ag_inflight = {}
        for j in range(S):
            for d_, rc in ((R, red_r), (L, red_l)):
                inflight[(d_, N_DEV - 2, j)].wait_recv()
                h_ref[srows(rc, d_, j), :] = (
                    h_ref[srows(rc, d_, j), :] + rs_comm[d_, N_DEV - 2, j]
                )
                rd = make_ag(d_, 0, j, h_ref.at[srows(rc, d_, j), :])
                rd.start()
                ag_inflight[(d_, 0, j)] = rd
        compute_out_rows(hrows(red_r, R))
        compute_out_rows(hrows(red_l, L))

        for t in range(1, N_DEV - 1):
            got_r = mod(my - (t - 1))
            got_l = mod(my + (t - 1))
            for j in range(S):
                for d_, gc in ((R, got_r), (L, got_l)):
                    ag_inflight[(d_, t - 1, j)].wait_recv()
                    rd = make_ag(d_, t, j, ag_comm.at[d_, t - 1, j])
                    rd.start()
                    ag_inflight[(d_, t, j)] = rd
                    h_ref[srows(gc, d_, j), :] = ag_comm[d_, t - 1, j]
                    compute_out_rows(srows(gc, d_, j))

        got_r = mod(my - 2)
        got_l = mod(my + 2)
        for j in range(S):
            for d_, gc in ((R, got_r), (L, got_l)):
                ag_inflight[(d_, N_DEV - 2, j)].wait_recv()
                h_ref[srows(gc, d_, j), :] = ag_comm[d_, N_DEV - 2, j]
                compute_out_rows(srows(gc, d_, j))

        for rd in inflight.values():
            rd.wait_send()
        for rd in ag_inflight.values():
            rd.wait_send()

    return pl.pallas_call(
        body,
        out_shape=jax.ShapeDtypeStruct((m, f), jnp.float32),
        in_specs=[
            pl.BlockSpec(memory_space=pltpu.VMEM),
            pl.BlockSpec(memory_space=pltpu.VMEM),
            pl.BlockSpec(memory_space=pltpu.VMEM),
        ],
        out_specs=pl.BlockSpec(memory_space=pltpu.VMEM),
        scratch_shapes=[
            pltpu.VMEM((m, d), jnp.float32),
            pltpu.VMEM((2, N_DEV - 1, S, chunk // 2 // S, d), jnp.float32),
            pltpu.VMEM((2, N_DEV - 1, S, chunk // 2 // S, d), jnp.float32),
            pltpu.SemaphoreType.DMA((2, N_DEV - 1, S)),
            pltpu.SemaphoreType.DMA((2, N_DEV - 1, S)),
            pltpu.SemaphoreType.DMA((2, N_DEV - 1, S)),
            pltpu.SemaphoreType.DMA((2, N_DEV - 1, S)),
        ],
        compiler_params=pltpu.CompilerParams(collective_id=0),
        input_output_aliases={0: 0},
    )(x, W1, W2)
